# Initial kernel scaffold; baseline (speedup 1.0000x reference)
import jax
import jax.numpy as jnp
from jax.experimental import pallas as pl


def kernel(target, window, non_adj_list, embed_table):
    raise NotImplementedError("write your pallas kernel here")


# speedup vs baseline: 2.3255x; 2.3255x over previous
import jax
import jax.numpy as jnp
from jax import lax
from jax.experimental import pallas as pl
from jax.experimental.pallas import tpu as pltpu
from jax.experimental.pallas import tpu_sc as plsc

NC, NS, L = 2, 16, 16
NW = NC * NS

B, WL, D = 4096, 20, 128
BW = B // NW
PAIRS = BW * WL
CHUNK = 128
NCHUNK = PAIRS // CHUNK


def _body(tgt_hbm, win_hbm, nadj_hbm, emb_hbm, out_hbm,
          tgt_idx, win_idx, widx_v, neg_idx, node_v, blk_v,
          tb0, pb0, nb0, tb1, pb1, nb1, acc_v,
          sem0, sem1, semh):
    wid = lax.axis_index("s") * NC + lax.axis_index("c")
    base = wid * PAIRS

    pltpu.sync_copy(tgt_hbm.at[pl.ds(base, PAIRS)], tgt_idx)
    pltpu.sync_copy(win_hbm.at[pl.ds(base, PAIRS)], win_idx)

    for i in range(BW // L):
        j = lax.iota(jnp.int32, L) + (i * L)
        p = j * WL + (WL - 1)
        node_v[pl.ds(i * L, L)] = plsc.load_gather(tgt_idx, [p])

    for i in range(2 * BW // L):
        j = lax.iota(jnp.int32, L) + (i * L)
        n = plsc.load_gather(node_v, [j >> 1])
        widx_v[pl.ds(i * L, L)] = ((n * WL) >> 4) + (j & 1)

    head = [pltpu.async_copy(nadj_hbm.at[widx_v.at[pl.ds(h * CHUNK, CHUNK)]],
                             blk_v.at[pl.ds(h * CHUNK, CHUNK)], semh)
            for h in range(2 * BW // CHUNK)]
    for cp in head:
        cp.wait()

    wl_vec = jnp.full((L,), WL, jnp.int32)
    def _extract(k, carry):
        p = lax.iota(jnp.int32, L) + k * L
        b = lax.div(p, wl_vec)
        l = p - b * wl_vec
        n = plsc.load_gather(node_v, [b])
        off = ((n * WL) & 15) + l
        row = (b << 1) + (off >> 4)
        col = off & 15
        neg_idx[pl.ds(k * L, L)] = plsc.load_gather(blk_v, [row, col])
        return carry
    lax.fori_loop(0, PAIRS // L, _extract, jnp.int32(0))

    slots = ((tb0, pb0, nb0, sem0), (tb1, pb1, nb1, sem1))

    def start(c, slot):
        tb, pb, nb, sem = slot
        sl = pl.ds(c * CHUNK, CHUNK)
        return (
            pltpu.async_copy(emb_hbm.at[tgt_idx.at[sl]], tb, sem),
            pltpu.async_copy(emb_hbm.at[win_idx.at[sl]], pb, sem),
            pltpu.async_copy(emb_hbm.at[neg_idx.at[sl]], nb, sem),
        )

    def compute(slot, acc):
        tb, pb, nb, _ = slot
        def row(r, a):
            for q in range(D // L):
                sl = pl.ds(q * L, L)
                a = a + tb[r, sl] * (nb[r, sl] - pb[r, sl])
            return a
        return lax.fori_loop(0, CHUNK, row, acc)

    acc = jnp.zeros((L,), jnp.float32)
    pending = {0: start(0, slots[0])}
    for c in range(NCHUNK):
        if c + 1 < NCHUNK:
            pending[c + 1] = start(c + 1, slots[(c + 1) % 2])
        for cp in pending.pop(c):
            cp.wait()
        acc = compute(slots[c % 2], acc)

    acc_v[...] = acc
    pltpu.sync_copy(acc_v, out_hbm.at[wid])


def kernel(target, window, non_adj_list, embed_table):
    mesh = plsc.VectorSubcoreMesh(
        core_axis_name="c", subcore_axis_name="s",
        num_cores=NC, num_subcores=NS)
    partials = pl.kernel(
        _body,
        out_type=jax.ShapeDtypeStruct((NW, L), jnp.float32),
        mesh=mesh,
        compiler_params=pltpu.CompilerParams(
            needs_layout_passes=False, use_tc_tiling_on_sc=False),
        scratch_types=[
            pltpu.VMEM((PAIRS,), jnp.int32),
            pltpu.VMEM((PAIRS,), jnp.int32),
            pltpu.VMEM((2 * BW,), jnp.int32),
            pltpu.VMEM((PAIRS,), jnp.int32),
            pltpu.VMEM((BW,), jnp.int32),
            pltpu.VMEM((2 * BW, 16), jnp.int32),
            pltpu.VMEM((CHUNK, D), jnp.float32),
            pltpu.VMEM((CHUNK, D), jnp.float32),
            pltpu.VMEM((CHUNK, D), jnp.float32),
            pltpu.VMEM((CHUNK, D), jnp.float32),
            pltpu.VMEM((CHUNK, D), jnp.float32),
            pltpu.VMEM((CHUNK, D), jnp.float32),
            pltpu.VMEM((L,), jnp.float32),
            pltpu.SemaphoreType.DMA,
            pltpu.SemaphoreType.DMA,
            pltpu.SemaphoreType.DMA,
        ],
    )(target.reshape(-1), window.reshape(-1),
      non_adj_list.reshape(-1, 16), embed_table)
    return jnp.sum(partials)


# speedup vs baseline: 2.3417x; 1.0069x over previous
import jax
import jax.numpy as jnp
from jax import lax
from jax.experimental import pallas as pl
from jax.experimental.pallas import tpu as pltpu
from jax.experimental.pallas import tpu_sc as plsc

NC, NS, L = 2, 16, 16
NW = NC * NS

B, WL, D = 4096, 20, 128
BW = B // NW
PAIRS = BW * WL
CHUNK = 128
NCHUNK = PAIRS // CHUNK


def _body(tgt_hbm, win_hbm, nadj_hbm, emb_hbm, out_hbm,
          tgt_idx, win_idx, widx_v, neg_idx, node_v, blk_v,
          tb0, pb0, nb0, tb1, pb1, nb1, acc_v,
          sem0, sem1, semh):
    wid = lax.axis_index("s") * NC + lax.axis_index("c")
    base = wid * PAIRS

    pltpu.sync_copy(tgt_hbm.at[pl.ds(base, PAIRS)], tgt_idx)
    pltpu.sync_copy(win_hbm.at[pl.ds(base, PAIRS)], win_idx)

    slots = ((tb0, pb0, nb0, sem0), (tb1, pb1, nb1, sem1))

    def start_tw(c, slot):
        tb, pb, _, sem = slot
        sl = pl.ds(c * CHUNK, CHUNK)
        return (
            pltpu.async_copy(emb_hbm.at[tgt_idx.at[sl]], tb, sem),
            pltpu.async_copy(emb_hbm.at[win_idx.at[sl]], pb, sem),
        )

    def start_neg(c, slot):
        _, _, nb, sem = slot
        sl = pl.ds(c * CHUNK, CHUNK)
        return (pltpu.async_copy(emb_hbm.at[neg_idx.at[sl]], nb, sem),)

    pending = {0: start_tw(0, slots[0]), 1: start_tw(1, slots[1])}

    for i in range(BW // L):
        j = lax.iota(jnp.int32, L) + (i * L)
        p = j * WL + (WL - 1)
        node_v[pl.ds(i * L, L)] = plsc.load_gather(tgt_idx, [p])

    for i in range(2 * BW // L):
        j = lax.iota(jnp.int32, L) + (i * L)
        n = plsc.load_gather(node_v, [j >> 1])
        widx_v[pl.ds(i * L, L)] = ((n * WL) >> 4) + (j & 1)

    head = [pltpu.async_copy(nadj_hbm.at[widx_v.at[pl.ds(h * CHUNK, CHUNK)]],
                             blk_v.at[pl.ds(h * CHUNK, CHUNK)], semh)
            for h in range(2 * BW // CHUNK)]
    for cp in head:
        cp.wait()

    wl_vec = jnp.full((L,), WL, jnp.int32)
    def _extract(k, carry):
        p = lax.iota(jnp.int32, L) + k * L
        b = lax.div(p, wl_vec)
        l = p - b * wl_vec
        n = plsc.load_gather(node_v, [b])
        off = ((n * WL) & 15) + l
        row = (b << 1) + (off >> 4)
        col = off & 15
        neg_idx[pl.ds(k * L, L)] = plsc.load_gather(blk_v, [row, col])
        return carry
    lax.fori_loop(0, PAIRS // L, _extract, jnp.int32(0))

    pending[0] += start_neg(0, slots[0])
    pending[1] += start_neg(1, slots[1])

    def compute(slot, acc):
        tb, pb, nb, _ = slot
        def row(r, a):
            t = []
            for q in range(D // L):
                sl = pl.ds(q * L, L)
                t.append(tb[r, sl] * (nb[r, sl] - pb[r, sl]))
            while len(t) > 1:
                t = [t[i] + t[i + 1] for i in range(0, len(t) - 1, 2)] \
                    + ([t[-1]] if len(t) % 2 else [])
            return a + t[0]
        return plsc.parallel_loop(0, CHUNK, unroll=2, carry=acc)(row)

    acc = jnp.zeros((L,), jnp.float32)
    for c in range(NCHUNK):
        for cp in pending.pop(c):
            cp.wait()
        acc = compute(slots[c % 2], acc)
        if c + 2 < NCHUNK:
            pending[c + 2] = (start_tw(c + 2, slots[c % 2])
                              + start_neg(c + 2, slots[c % 2]))

    acc_v[...] = acc
    pltpu.sync_copy(acc_v, out_hbm.at[wid])


def kernel(target, window, non_adj_list, embed_table):
    mesh = plsc.VectorSubcoreMesh(
        core_axis_name="c", subcore_axis_name="s",
        num_cores=NC, num_subcores=NS)
    partials = pl.kernel(
        _body,
        out_type=jax.ShapeDtypeStruct((NW, L), jnp.float32),
        mesh=mesh,
        compiler_params=pltpu.CompilerParams(
            needs_layout_passes=False, use_tc_tiling_on_sc=False),
        scratch_types=[
            pltpu.VMEM((PAIRS,), jnp.int32),
            pltpu.VMEM((PAIRS,), jnp.int32),
            pltpu.VMEM((2 * BW,), jnp.int32),
            pltpu.VMEM((PAIRS,), jnp.int32),
            pltpu.VMEM((BW,), jnp.int32),
            pltpu.VMEM((2 * BW, 16), jnp.int32),
            pltpu.VMEM((CHUNK, D), jnp.float32),
            pltpu.VMEM((CHUNK, D), jnp.float32),
            pltpu.VMEM((CHUNK, D), jnp.float32),
            pltpu.VMEM((CHUNK, D), jnp.float32),
            pltpu.VMEM((CHUNK, D), jnp.float32),
            pltpu.VMEM((CHUNK, D), jnp.float32),
            pltpu.VMEM((L,), jnp.float32),
            pltpu.SemaphoreType.DMA,
            pltpu.SemaphoreType.DMA,
            pltpu.SemaphoreType.DMA,
        ],
    )(target.reshape(-1), window.reshape(-1),
      non_adj_list.reshape(-1, 16), embed_table)
    return jnp.sum(partials)
